# baseline (speedup 1.0000x reference)
import functools

import jax
import jax.numpy as jnp
from jax import lax
from jax.experimental import pallas as pl
from jax.experimental.pallas import tpu as pltpu
from jax.experimental.pallas import tpu_sc as plsc

B = 1024
K = 512
D = 256
N = 50000
S = 4

L = 16
NC = 2
NS = 16
NW = NC * NS

CH = 16
NCHUNKS = N // CH
SEGW = B // NW
KD = D // L

_mesh = plsc.VectorSubcoreMesh(core_axis_name="c", subcore_axis_name="s")


def _tree_sum(parts):
    while len(parts) > 1:
        parts = [parts[j] + parts[j + 1] for j in range(0, len(parts) - 1, 2)] \
            + ([parts[-1]] if len(parts) % 2 else [])
    return parts[0]


@functools.partial(
    pl.kernel,
    mesh=_mesh,
    out_type=jax.ShapeDtypeStruct((B, D), jnp.float32),
    scratch_types=[
        pltpu.VMEM((N + L,), jnp.int32),
        pltpu.VMEM((2, CH, S, D), jnp.float32),
        pltpu.VMEM((KD, L), jnp.float32),
        pltpu.VMEM((SEGW + 1, D), jnp.float32),
        pltpu.VMEM((SEGW + 1, L), jnp.float32),
        pltpu.VMEM((SEGW, D), jnp.float32),
        pltpu.SMEM((8,), jnp.int32),
        pltpu.SMEM((8,), jnp.float32),
        pltpu.SemaphoreType.DMA,
        pltpu.SemaphoreType.DMA,
    ],
)
def _sc_segment_mean(nd_hbm, bi_hbm, out_hbm, bi_v, buf, runbuf, acc, cnt,
                     trad, sreg_i, sreg_f, sem_a, sem_b):
    cid = lax.axis_index("c")
    sid = lax.axis_index("s")
    wid = sid * NC + cid
    base = wid * SEGW

    pltpu.sync_copy(bi_hbm, bi_v.at[pl.ds(0, N)])

    for r in range(SEGW + 1):
        for k in range(KD):
            acc[r, pl.ds(k * L, L)] = jnp.zeros((L,), jnp.float32)
        cnt[r, :] = jnp.zeros((L,), jnp.float32)
    for k in range(KD):
        runbuf[k, :] = jnp.zeros((L,), jnp.float32)
    sreg_i[0] = jnp.int32(-1)
    sreg_f[0] = jnp.float32(0.0)

    def lower_bound(t):
        pos = jnp.int32(0)
        for sh in range(15, -1, -1):
            nxt = pos + (1 << sh)
            probe = bi_v[pl.ds(jnp.minimum(nxt - 1, N - 1), L)][0]
            ok = (nxt <= N) & (probe < t)
            pos = jnp.where(ok, nxt, pos)
        return pos

    lo = lower_bound(base)
    hi = lower_bound(base + SEGW)
    m0 = lax.div(lo, CH)
    m1 = lax.div(hi + (CH - 1), CH)
    total = m1 - m0

    def start(m, bufp, sem):
        pltpu.async_copy(nd_hbm.at[pl.ds(pl.multiple_of(m * CH, CH), CH)],
                         bufp, sem)

    def wait(bufp, sem):
        pltpu.make_async_copy(nd_hbm.at[pl.ds(0, CH)], bufp, sem).wait()

    def flush_run():
        run_id = sreg_i[0]
        ok = (run_id >= base) & (run_id < base + SEGW)
        r = jnp.where(ok, run_id - base, SEGW)
        for k in range(KD):
            acc[r, pl.ds(k * L, L)] += runbuf[k, :]
        cnt[r, :] += jnp.broadcast_to(sreg_f[0], (L,))

    def process(m, bufp):
        node0 = pl.multiple_of(m * CH, CH)
        ids = bi_v[pl.ds(node0, CH)]
        rid = sreg_i[0]
        fastok = ((ids[0] == rid) & (ids[CH - 1] == rid)
                  & (node0 >= lo) & (node0 + CH <= hi))

        @pl.when(fastok)
        def _():
            for k in range(KD):
                sl = pl.ds(k * L, L)
                tot = _tree_sum([
                    (bufp[i, 0, sl] + bufp[i, 1, sl])
                    + (bufp[i, 2, sl] + bufp[i, 3, sl])
                    for i in range(CH)
                ])
                runbuf[k, :] += tot
            sreg_f[0] = sreg_f[0] + float(CH)

        @pl.when(jnp.logical_not(fastok))
        def _():
            for i in range(CH):
                gi = node0 + i
                in_range = (gi >= lo) & (gi < hi)
                nid = jnp.where(in_range, ids[i], jnp.int32(-1))

                @pl.when(nid != sreg_i[0])
                def _():
                    flush_run()
                    sreg_f[0] = jnp.float32(0.0)
                    sreg_i[0] = nid
                    for k in range(KD):
                        runbuf[k, :] = jnp.zeros((L,), jnp.float32)

                for k in range(KD):
                    sl = pl.ds(k * L, L)
                    runbuf[k, :] += ((bufp[i, 0, sl] + bufp[i, 1, sl])
                                     + (bufp[i, 2, sl] + bufp[i, 3, sl]))
                sreg_f[0] = sreg_f[0] + 1.0

    @pl.when(total > 0)
    def _():
        start(m0, buf.at[0], sem_a)

    def body(g, carry):
        m = m0 + g
        p = lax.rem(g, 2)

        @pl.when((p == 0) & (g + 1 < total))
        def _():
            start(m + 1, buf.at[1], sem_b)

        @pl.when((p == 1) & (g + 1 < total))
        def _():
            start(m + 1, buf.at[0], sem_a)

        @pl.when(p == 0)
        def _():
            wait(buf.at[0], sem_a)

        @pl.when(p == 1)
        def _():
            wait(buf.at[1], sem_b)

        process(m, buf.at[p])
        return carry

    lax.fori_loop(0, total, body, 0)
    flush_run()

    for r in range(SEGW):
        dv = jnp.maximum(cnt[r, :] * float(S), 1.0)
        for k in range(KD):
            sl = pl.ds(k * L, L)
            trad[r, sl] = acc[r, sl] / dv
    pltpu.sync_copy(trad, out_hbm.at[pl.ds(base, SEGW)])


def _tc_matmul_body(bw_ref, cb_ref, o_ref):
    o_ref[...] = jnp.dot(bw_ref[...], cb_ref[...],
                         preferred_element_type=jnp.float32)


_tc_matmul = pl.pallas_call(
    _tc_matmul_body,
    out_shape=jax.ShapeDtypeStruct((B, D), jnp.float32),
)


def kernel(barycenter_weights, codebook, node_distributions, batch_idx):
    bi = batch_idx.astype(jnp.int32)
    trad = _sc_segment_mean(node_distributions, bi)
    mm = _tc_matmul(barycenter_weights, codebook)
    return jnp.concatenate([mm, trad], axis=1)


# speedup vs baseline: 1.9406x; 1.9406x over previous
import functools

import jax
import jax.numpy as jnp
from jax import lax
from jax.experimental import pallas as pl
from jax.experimental.pallas import tpu as pltpu
from jax.experimental.pallas import tpu_sc as plsc

B = 1024
K = 512
D = 256
N = 50000
S = 4

L = 16
NC = 2
NS = 16
NW = NC * NS

CH = 16
NCHUNKS = N // CH
SEGW = B // NW
KD = D // L

_mesh = plsc.VectorSubcoreMesh(core_axis_name="c", subcore_axis_name="s")


def _tree_sum(parts):
    while len(parts) > 1:
        parts = [parts[j] + parts[j + 1] for j in range(0, len(parts) - 1, 2)] \
            + ([parts[-1]] if len(parts) % 2 else [])
    return parts[0]


@functools.partial(
    pl.kernel,
    mesh=_mesh,
    out_type=jax.ShapeDtypeStruct((B, D), jnp.float32),
    scratch_types=[
        pltpu.VMEM((N + 2 * L,), jnp.int32),
        pltpu.VMEM((2, CH, S, D), jnp.float32),
        pltpu.VMEM((KD, L), jnp.float32),
        pltpu.VMEM((SEGW + 1, D), jnp.float32),
        pltpu.VMEM((SEGW + 1, L), jnp.float32),
        pltpu.VMEM((SEGW, D), jnp.float32),
        pltpu.SMEM((8,), jnp.int32),
        pltpu.SMEM((8,), jnp.float32),
        pltpu.SemaphoreType.DMA,
        pltpu.SemaphoreType.DMA,
    ],
)
def _sc_segment_mean(nd_hbm, bi_hbm, out_hbm, bi_v, buf, runbuf, acc, cnt,
                     trad, sreg_i, sreg_f, sem_a, sem_b):
    cid = lax.axis_index("c")
    sid = lax.axis_index("s")
    wid = sid * NC + cid
    base = wid * SEGW

    pltpu.sync_copy(bi_hbm, bi_v.at[pl.ds(0, N)])

    for r in range(SEGW + 1):
        for k in range(KD):
            acc[r, pl.ds(k * L, L)] = jnp.zeros((L,), jnp.float32)
        cnt[r, :] = jnp.zeros((L,), jnp.float32)
    for k in range(KD):
        runbuf[k, :] = jnp.zeros((L,), jnp.float32)
    sreg_i[0] = jnp.int32(-1)
    sreg_f[0] = jnp.float32(0.0)

    def lower_bound(t):
        pos = jnp.int32(0)
        for sh in range(15, -1, -1):
            nxt = pos + (1 << sh)
            probe = bi_v[pl.ds(jnp.minimum(nxt - 1, N - 1), L)][0]
            ok = (nxt <= N) & (probe < t)
            pos = jnp.where(ok, nxt, pos)
        return pos

    lo = lower_bound(base)
    hi = lower_bound(base + SEGW)
    m0 = lax.div(lo, CH)
    m1 = lax.div(hi + (CH - 1), CH)
    total = m1 - m0

    def start(m, bufp, sem):
        pltpu.async_copy(nd_hbm.at[pl.ds(pl.multiple_of(m * CH, CH), CH)],
                         bufp, sem)

    def wait(bufp, sem):
        pltpu.make_async_copy(nd_hbm.at[pl.ds(0, CH)], bufp, sem).wait()

    def flush_run():
        run_id = sreg_i[0]
        ok = (run_id >= base) & (run_id < base + SEGW)
        r = jnp.where(ok, run_id - base, SEGW)
        for k in range(KD):
            acc[r, pl.ds(k * L, L)] += runbuf[k, :]
        cnt[r, :] += jnp.broadcast_to(sreg_f[0], (L,))

    def process(m, bufp):
        node0 = pl.multiple_of(m * CH, CH)
        ids = bi_v[pl.ds(node0, CH)]
        rid = sreg_i[0]
        fastok = ((ids[0] == rid) & (ids[CH - 1] == rid)
                  & (node0 >= lo) & (node0 + CH <= hi))

        @pl.when(fastok)
        def _():
            for k in range(KD):
                sl = pl.ds(k * L, L)
                tot = _tree_sum([
                    (bufp[i, 0, sl] + bufp[i, 1, sl])
                    + (bufp[i, 2, sl] + bufp[i, 3, sl])
                    for i in range(CH)
                ])
                runbuf[k, :] += tot
            sreg_f[0] = sreg_f[0] + float(CH)

        @pl.when(jnp.logical_not(fastok))
        def _():
            def node_body(i, carry):
                gi = node0 + i
                in_range = (gi >= lo) & (gi < hi)
                nid0 = bi_v[pl.ds(gi, L)][0]
                nid = jnp.where(in_range, nid0, jnp.int32(-1))

                @pl.when(nid != sreg_i[0])
                def _():
                    flush_run()
                    sreg_f[0] = jnp.float32(0.0)
                    sreg_i[0] = nid
                    for k in range(KD):
                        runbuf[k, :] = jnp.zeros((L,), jnp.float32)

                for k in range(KD):
                    sl = pl.ds(k * L, L)
                    runbuf[k, :] += ((bufp[i, 0, sl] + bufp[i, 1, sl])
                                     + (bufp[i, 2, sl] + bufp[i, 3, sl]))
                sreg_f[0] = sreg_f[0] + 1.0
                return carry

            lax.fori_loop(0, CH, node_body, 0)

    @pl.when(total > 0)
    def _():
        start(m0, buf.at[0], sem_a)

    def body(g, carry):
        m = m0 + g
        p = lax.rem(g, 2)

        @pl.when((p == 0) & (g + 1 < total))
        def _():
            start(m + 1, buf.at[1], sem_b)

        @pl.when((p == 1) & (g + 1 < total))
        def _():
            start(m + 1, buf.at[0], sem_a)

        @pl.when(p == 0)
        def _():
            wait(buf.at[0], sem_a)

        @pl.when(p == 1)
        def _():
            wait(buf.at[1], sem_b)

        process(m, buf.at[p])
        return carry

    lax.fori_loop(0, total, body, 0)
    flush_run()

    for r in range(SEGW):
        dv = jnp.maximum(cnt[r, :] * float(S), 1.0)
        for k in range(KD):
            sl = pl.ds(k * L, L)
            trad[r, sl] = acc[r, sl] / dv
    pltpu.sync_copy(trad, out_hbm.at[pl.ds(base, SEGW)])


def _tc_matmul_body(bw_ref, cb_ref, o_ref):
    o_ref[...] = jnp.dot(bw_ref[...], cb_ref[...],
                         preferred_element_type=jnp.float32)


_tc_matmul = pl.pallas_call(
    _tc_matmul_body,
    out_shape=jax.ShapeDtypeStruct((B, D), jnp.float32),
)


def kernel(barycenter_weights, codebook, node_distributions, batch_idx):
    bi = batch_idx.astype(jnp.int32)
    trad = _sc_segment_mean(node_distributions, bi)
    mm = _tc_matmul(barycenter_weights, codebook)
    return jnp.concatenate([mm, trad], axis=1)


# speedup vs baseline: 2.0739x; 1.0687x over previous
import functools

import jax
import jax.numpy as jnp
from jax import lax
from jax.experimental import pallas as pl
from jax.experimental.pallas import tpu as pltpu
from jax.experimental.pallas import tpu_sc as plsc

B = 1024
K = 512
D = 256
N = 50000
S = 4

L = 16
NC = 2
NS = 16
NW = NC * NS

CH = 8
NCHUNKS = N // CH
SEGW = B // NW
KD = D // L

_mesh = plsc.VectorSubcoreMesh(core_axis_name="c", subcore_axis_name="s")


def _tree_sum(parts):
    while len(parts) > 1:
        parts = [parts[j] + parts[j + 1] for j in range(0, len(parts) - 1, 2)] \
            + ([parts[-1]] if len(parts) % 2 else [])
    return parts[0]


@functools.partial(
    pl.kernel,
    mesh=_mesh,
    out_type=jax.ShapeDtypeStruct((B, D), jnp.float32),
    scratch_types=[
        pltpu.VMEM((N + 2 * L,), jnp.int32),
        pltpu.VMEM((2, CH, S, D), jnp.float32),
        pltpu.VMEM((KD, L), jnp.float32),
        pltpu.VMEM((SEGW + 1, D), jnp.float32),
        pltpu.VMEM((SEGW + 1, L), jnp.float32),
        pltpu.VMEM((SEGW, D), jnp.float32),
        pltpu.SMEM((8,), jnp.int32),
        pltpu.SMEM((8,), jnp.float32),
        pltpu.SemaphoreType.DMA,
        pltpu.SemaphoreType.DMA,
    ],
)
def _sc_segment_mean(nd_hbm, bi_hbm, out_hbm, bi_v, buf, runbuf, acc, cnt,
                     trad, sreg_i, sreg_f, sem_a, sem_b):
    cid = lax.axis_index("c")
    sid = lax.axis_index("s")
    wid = sid * NC + cid
    base = wid * SEGW

    pltpu.sync_copy(bi_hbm, bi_v.at[pl.ds(0, N)])

    for r in range(SEGW + 1):
        for k in range(KD):
            acc[r, pl.ds(k * L, L)] = jnp.zeros((L,), jnp.float32)
        cnt[r, :] = jnp.zeros((L,), jnp.float32)
    for k in range(KD):
        runbuf[k, :] = jnp.zeros((L,), jnp.float32)
    sreg_i[0] = jnp.int32(-1)
    sreg_f[0] = jnp.float32(0.0)

    def lower_bound(t):
        pos = jnp.int32(0)
        for sh in range(15, -1, -1):
            nxt = pos + (1 << sh)
            probe = bi_v[pl.ds(jnp.minimum(nxt - 1, N - 1), L)][0]
            ok = (nxt <= N) & (probe < t)
            pos = jnp.where(ok, nxt, pos)
        return pos

    lo = lower_bound(base)
    hi = lower_bound(base + SEGW)
    m0 = lax.div(lo, CH)
    m1 = lax.div(hi + (CH - 1), CH)
    total = m1 - m0

    def start(m, bufp, sem):
        pltpu.async_copy(nd_hbm.at[pl.ds(pl.multiple_of(m * CH, CH), CH)],
                         bufp, sem)

    def wait(bufp, sem):
        pltpu.make_async_copy(nd_hbm.at[pl.ds(0, CH)], bufp, sem).wait()

    def flush_run():
        run_id = sreg_i[0]
        ok = (run_id >= base) & (run_id < base + SEGW)
        r = jnp.where(ok, run_id - base, SEGW)
        for k in range(KD):
            acc[r, pl.ds(k * L, L)] += runbuf[k, :]
        cnt[r, :] += jnp.broadcast_to(sreg_f[0], (L,))

    def process(m, bufp):
        node0 = pl.multiple_of(m * CH, CH)
        ids = bi_v[pl.ds(node0, L)]
        rid = sreg_i[0]
        fastok = ((ids[0] == rid) & (ids[CH - 1] == rid)
                  & (node0 >= lo) & (node0 + CH <= hi))

        @pl.when(fastok)
        def _():
            for k in range(KD):
                sl = pl.ds(k * L, L)
                tot = _tree_sum([
                    (bufp[i, 0, sl] + bufp[i, 1, sl])
                    + (bufp[i, 2, sl] + bufp[i, 3, sl])
                    for i in range(CH)
                ])
                runbuf[k, :] += tot
            sreg_f[0] = sreg_f[0] + float(CH)

        @pl.when(jnp.logical_not(fastok))
        def _():
            def node_body(i, carry):
                gi = node0 + i
                in_range = (gi >= lo) & (gi < hi)
                nid0 = bi_v[pl.ds(gi, L)][0]
                nid = jnp.where(in_range, nid0, jnp.int32(-1))

                @pl.when(nid != sreg_i[0])
                def _():
                    flush_run()
                    sreg_f[0] = jnp.float32(0.0)
                    sreg_i[0] = nid
                    for k in range(KD):
                        runbuf[k, :] = jnp.zeros((L,), jnp.float32)

                for k in range(KD):
                    sl = pl.ds(k * L, L)
                    runbuf[k, :] += ((bufp[i, 0, sl] + bufp[i, 1, sl])
                                     + (bufp[i, 2, sl] + bufp[i, 3, sl]))
                sreg_f[0] = sreg_f[0] + 1.0
                return carry

            lax.fori_loop(0, CH, node_body, 0)

    @pl.when(total > 0)
    def _():
        start(m0, buf.at[0], sem_a)

    def body(g, carry):
        m = m0 + g
        p = lax.rem(g, 2)

        @pl.when((p == 0) & (g + 1 < total))
        def _():
            start(m + 1, buf.at[1], sem_b)

        @pl.when((p == 1) & (g + 1 < total))
        def _():
            start(m + 1, buf.at[0], sem_a)

        @pl.when(p == 0)
        def _():
            wait(buf.at[0], sem_a)

        @pl.when(p == 1)
        def _():
            wait(buf.at[1], sem_b)

        process(m, buf.at[p])
        return carry

    lax.fori_loop(0, total, body, 0)
    flush_run()

    for r in range(SEGW):
        dv = jnp.maximum(cnt[r, :] * float(S), 1.0)
        for k in range(KD):
            sl = pl.ds(k * L, L)
            trad[r, sl] = acc[r, sl] / dv
    pltpu.sync_copy(trad, out_hbm.at[pl.ds(base, SEGW)])


def _tc_matmul_body(bw_ref, cb_ref, o_ref):
    o_ref[...] = jnp.dot(bw_ref[...], cb_ref[...],
                         preferred_element_type=jnp.float32)


_tc_matmul = pl.pallas_call(
    _tc_matmul_body,
    out_shape=jax.ShapeDtypeStruct((B, D), jnp.float32),
)


def kernel(barycenter_weights, codebook, node_distributions, batch_idx):
    bi = batch_idx.astype(jnp.int32)
    trad = _sc_segment_mean(node_distributions, bi)
    mm = _tc_matmul(barycenter_weights, codebook)
    return jnp.concatenate([mm, trad], axis=1)
